# baseline (speedup 1.0000x reference)
import jax
import jax.numpy as jnp
from jax import lax
from jax.experimental import pallas as pl
from jax.experimental.pallas import tpu as pltpu
from jax.experimental.pallas import tpu_sc as plsc

NG = 32768
NM = 10242
DF = 128
HD = 128
EG2M = 163840
EMESH = 81920
EM2G = 98304

NC = 2
NS = 16
NW = NC * NS
CH = 128

NMP = 10368
ZRM = NMP // NS

MGC = 8192
MACC = MGC + 128
ZRG = MACC // NS
ORG = MGC // NS

_MESH = plsc.VectorSubcoreMesh(core_axis_name="c", subcore_axis_name="s")


def _pieces(total):
  out = []
  off = 0
  while off < total:
    n = min(CH, total - off)
    out.append((off, n))
    off += n
  return out


def _zero_spmem(stage, acc, row0, nrows):
  for off, n in _pieces(nrows):
    pltpu.sync_copy(stage.at[pl.ds(0, n)], acc.at[pl.ds(row0 + off, n)])


def _spmem_to_hbm(acc, stage, out_at, row0, nrows):
  for off, n in _pieces(nrows):
    pltpu.sync_copy(acc.at[pl.ds(row0 + off, n)], stage.at[pl.ds(0, n)])
    pltpu.sync_copy(stage.at[pl.ds(0, n)], out_at(off, n))


def _make_segsum(n_edges):
  epw = n_edges // NW
  nch = epw // CH

  def body(table, src, dst, zrows, out, acc, idx_s, idx_d, rows, sem):
    cid = lax.axis_index("c")
    sid = lax.axis_index("s")
    wid = sid * NC + cid
    r0 = sid * ZRM
    pltpu.sync_copy(zrows, rows)
    _zero_spmem(rows, acc, r0, ZRM)
    plsc.subcore_barrier()

    base = wid * epw

    def step(j, carry):
      o = base + j * CH
      pltpu.sync_copy(src.at[pl.ds(o, CH)], idx_s)
      pltpu.async_copy(table.at[idx_s], rows, sem).wait()
      pltpu.sync_copy(dst.at[pl.ds(o, CH)], idx_d)
      pltpu.sync_copy(rows, acc.at[idx_d], add=True)
      return carry

    lax.fori_loop(0, nch, step, 0, unroll=False)
    plsc.subcore_barrier()
    _spmem_to_hbm(acc, rows,
                  lambda off, n: out.at[cid, pl.ds(r0 + off, n)], r0, ZRM)

  return pl.kernel(
      body,
      out_type=jax.ShapeDtypeStruct((NC, NMP, HD), jnp.float32),
      mesh=_MESH,
      scratch_types=[
          pltpu.VMEM_SHARED((NMP, HD), jnp.float32),
          pltpu.VMEM((CH,), jnp.int32),
          pltpu.VMEM((CH,), jnp.int32),
          pltpu.VMEM((CH, HD), jnp.float32),
          pltpu.SemaphoreType.DMA,
      ])


def _make_deg_mesh(n_edges):
  epw = n_edges // NW
  nch = epw // CH

  def body(dst, zrows, ones, out, acc, idx_d, buf):
    cid = lax.axis_index("c")
    sid = lax.axis_index("s")
    wid = sid * NC + cid
    r0 = sid * ZRM
    pltpu.sync_copy(zrows, buf)
    _zero_spmem(buf, acc, r0, ZRM)
    pltpu.sync_copy(ones, buf)
    plsc.subcore_barrier()

    base = wid * epw

    def step(j, carry):
      o = base + j * CH
      pltpu.sync_copy(dst.at[pl.ds(o, CH)], idx_d)
      pltpu.sync_copy(buf, acc.at[idx_d], add=True)
      return carry

    lax.fori_loop(0, nch, step, 0, unroll=False)
    plsc.subcore_barrier()
    _spmem_to_hbm(acc, buf,
                  lambda off, n: out.at[cid, pl.ds(r0 + off, n)], r0, ZRM)

  return pl.kernel(
      body,
      out_type=jax.ShapeDtypeStruct((NC, NMP, HD), jnp.float32),
      mesh=_MESH,
      scratch_types=[
          pltpu.VMEM_SHARED((NMP, HD), jnp.float32),
          pltpu.VMEM((CH,), jnp.int32),
          pltpu.VMEM((CH, HD), jnp.float32),
      ])


def _make_m2g(with_gather):
  ept = EM2G // NS
  nch = ept // CH

  def body_sum(table, esrc, eidx, zrows, out, acc, idx_s, idx_d, rows,
               sem):
    cid = lax.axis_index("c")
    sid = lax.axis_index("s")
    base = sid * ept

    for cc in range(2):
      chunk = cid * 2 + cc
      pltpu.sync_copy(zrows, rows)
      _zero_spmem(rows, acc, sid * ZRG, ZRG)
      plsc.subcore_barrier()

      def step(j, carry):
        o = base + j * CH
        pltpu.sync_copy(esrc.at[chunk, pl.ds(o, CH)], idx_s)
        pltpu.async_copy(table.at[idx_s], rows, sem).wait()
        pltpu.sync_copy(eidx.at[chunk, pl.ds(o, CH)], idx_d)
        pltpu.sync_copy(rows, acc.at[idx_d], add=True)
        return carry

      lax.fori_loop(0, nch, step, 0, unroll=False)
      plsc.subcore_barrier()
      lo = chunk * MGC + sid * ORG
      _spmem_to_hbm(acc, rows,
                    lambda off, n: out.at[pl.ds(lo + off, n)],
                    sid * ORG, ORG)
      plsc.subcore_barrier()

  def body_deg(eidx, zrows, ones, out, acc, idx_d, buf):
    cid = lax.axis_index("c")
    sid = lax.axis_index("s")
    base = sid * ept

    for cc in range(2):
      chunk = cid * 2 + cc
      pltpu.sync_copy(zrows, buf)
      _zero_spmem(buf, acc, sid * ZRG, ZRG)
      pltpu.sync_copy(ones, buf)
      plsc.subcore_barrier()

      def step(j, carry):
        o = base + j * CH
        pltpu.sync_copy(eidx.at[chunk, pl.ds(o, CH)], idx_d)
        pltpu.sync_copy(buf, acc.at[idx_d], add=True)
        return carry

      lax.fori_loop(0, nch, step, 0, unroll=False)
      plsc.subcore_barrier()
      lo = chunk * MGC + sid * ORG
      _spmem_to_hbm(acc, buf,
                    lambda off, n: out.at[pl.ds(lo + off, n)],
                    sid * ORG, ORG)
      plsc.subcore_barrier()

  if with_gather:
    return pl.kernel(
        body_sum,
        out_type=jax.ShapeDtypeStruct((NG, HD), jnp.float32),
        mesh=_MESH,
        scratch_types=[
            pltpu.VMEM_SHARED((MACC, HD), jnp.float32),
            pltpu.VMEM((CH,), jnp.int32),
            pltpu.VMEM((CH,), jnp.int32),
            pltpu.VMEM((CH, HD), jnp.float32),
            pltpu.SemaphoreType.DMA,
        ])
  return pl.kernel(
      body_deg,
      out_type=jax.ShapeDtypeStruct((NG, HD), jnp.float32),
      mesh=_MESH,
      scratch_types=[
          pltpu.VMEM_SHARED((MACC, HD), jnp.float32),
          pltpu.VMEM((CH,), jnp.int32),
          pltpu.VMEM((CH, HD), jnp.float32),
      ])


def _enc_body(x_ref, w_ref, b_ref, o_ref):
  o_ref[...] = jnp.maximum(
      jnp.dot(x_ref[...], w_ref[...], preferred_element_type=jnp.float32)
      + b_ref[...], 0.0)


def _mesh_enc_body(agg_ref, deg_ref, wg_ref, wm_ref, b_ref, o_ref):
  s = agg_ref[0] + agg_ref[1]
  d = deg_ref[0, :, :1] + deg_ref[1, :, :1]
  h = s / jnp.maximum(d, 1.0)
  w = jnp.dot(wg_ref[...], wm_ref[...], preferred_element_type=jnp.float32)
  o_ref[...] = jnp.maximum(
      jnp.dot(h, w, preferred_element_type=jnp.float32) + b_ref[...], 0.0)


def _proc_body(h_ref, a_ref, w_ref, b_ref, o_ref):
  a = a_ref[0] + a_ref[1]
  o_ref[...] = h_ref[...] + jnp.maximum(
      jnp.dot(a, w_ref[...], preferred_element_type=jnp.float32)
      + b_ref[...], 0.0)


def _dec_body(hg_ref, a_ref, deg_ref, w1_ref, b1_ref, w2_ref, b2_ref,
              o_ref):
  d = jnp.maximum(deg_ref[:, :1], 1.0)
  t = jnp.maximum(
      jnp.dot(hg_ref[...], w1_ref[0], preferred_element_type=jnp.float32)
      + jnp.dot(a_ref[...] / d, w1_ref[1],
                preferred_element_type=jnp.float32)
      + b1_ref[...], 0.0)
  o_ref[...] = jnp.dot(t, w2_ref[...],
                       preferred_element_type=jnp.float32) + b2_ref[...]


_segsum_g2m = _make_segsum(EG2M)
_segsum_mesh = _make_segsum(EMESH)
_deg_mesh = _make_deg_mesh(EG2M)
_m2g_sum = _make_m2g(True)
_m2g_deg = _make_m2g(False)


@jax.jit
def kernel(X, g2m_src, g2m_dst, mesh_edge_index, m2g_src, m2g_dst, W_ge,
           b_ge, W_g2m, W_me, b_me, W_p1, b_p1, W_p2, b_p2, W_d1, b_d1,
           W_d2, b_d2):
  x = X[0]
  b_ge2 = b_ge[None]
  b_me2 = b_me[None]
  b_d12 = b_d1[None]
  b_d22 = b_d2[None]
  w_d1s = W_d1.reshape(2, HD, HD)

  zrows = jnp.zeros((CH, HD), jnp.float32)
  ones_r = jnp.ones((CH, HD), jnp.float32)

  spread = jnp.arange(EM2G, dtype=jnp.int32) & 127
  chunk_of = lax.shift_right_logical(m2g_dst, 13)
  local = jnp.bitwise_and(m2g_dst, MGC - 1)
  in_chunk = chunk_of[None, :] == jnp.arange(4, dtype=jnp.int32)[:, None]
  eidx = jnp.where(in_chunk, local[None, :], MGC + spread[None, :])
  esrc = jnp.where(in_chunk, m2g_src[None, :], 0)

  nb = 8
  h_grid = pl.pallas_call(
      _enc_body,
      grid=(nb,),
      in_specs=[
          pl.BlockSpec((NG // nb, DF), lambda i: (i, 0)),
          pl.BlockSpec((DF, HD), lambda i: (0, 0)),
          pl.BlockSpec((1, HD), lambda i: (0, 0)),
      ],
      out_specs=pl.BlockSpec((NG // nb, HD), lambda i: (i, 0)),
      out_shape=jax.ShapeDtypeStruct((NG, HD), jnp.float32),
  )(x, W_ge, b_ge2)

  agg = _segsum_g2m(h_grid, g2m_src, g2m_dst, zrows)
  deg = _deg_mesh(g2m_dst, zrows, ones_r)

  h_mesh = pl.pallas_call(
      _mesh_enc_body,
      out_shape=jax.ShapeDtypeStruct((NMP, HD), jnp.float32),
  )(agg, deg, W_g2m, W_me, b_me2)

  msrc = mesh_edge_index[0]
  mdst = mesh_edge_index[1]
  for w, b in ((W_p1, b_p1[None]), (W_p2, b_p2[None])):
    a = _segsum_mesh(h_mesh, msrc, mdst, zrows)
    h_mesh = pl.pallas_call(
        _proc_body,
        out_shape=jax.ShapeDtypeStruct((NMP, HD), jnp.float32),
    )(h_mesh, a, w, b)

  a3 = _m2g_sum(h_mesh, esrc, eidx, zrows)
  deg_g = _m2g_deg(eidx, zrows, ones_r)

  out = pl.pallas_call(
      _dec_body,
      grid=(nb,),
      in_specs=[
          pl.BlockSpec((NG // nb, HD), lambda i: (i, 0)),
          pl.BlockSpec((NG // nb, HD), lambda i: (i, 0)),
          pl.BlockSpec((NG // nb, HD), lambda i: (i, 0)),
          pl.BlockSpec((2, HD, HD), lambda i: (0, 0, 0)),
          pl.BlockSpec((1, HD), lambda i: (0, 0)),
          pl.BlockSpec((HD, DF), lambda i: (0, 0)),
          pl.BlockSpec((1, DF), lambda i: (0, 0)),
      ],
      out_specs=pl.BlockSpec((NG // nb, DF), lambda i: (i, 0)),
      out_shape=jax.ShapeDtypeStruct((NG, DF), jnp.float32),
  )(h_grid, a3, deg_g, w_d1s, b_d12, W_d2, b_d22)

  return out[None]


# speedup vs baseline: 13.7347x; 13.7347x over previous
import jax
import jax.numpy as jnp
from jax import lax
from jax.experimental import pallas as pl
from jax.experimental.pallas import tpu as pltpu
from jax.experimental.pallas import tpu_sc as plsc

NG = 32768
NM = 10242
DF = 128
HD = 128
EG2M = 163840
EMESH = 81920
EM2G = 98304

NC = 2
NS = 16
NW = NC * NS
CH = 128

NMP = 10368
ZRM = NMP // NS

MGC = 8192
MACC = MGC + 128
ZRG = MACC // NS
ORG = MGC // NS

_MESH = plsc.VectorSubcoreMesh(core_axis_name="c", subcore_axis_name="s")


def _pieces(total):
  out = []
  off = 0
  while off < total:
    n = min(CH, total - off)
    out.append((off, n))
    off += n
  return out


def _zero_spmem(stage, acc, row0, nrows):
  for off, n in _pieces(nrows):
    pltpu.sync_copy(stage.at[pl.ds(0, n)], acc.at[pl.ds(row0 + off, n)])


def _spmem_to_hbm(acc, stage, out_at, row0, nrows):
  for off, n in _pieces(nrows):
    pltpu.sync_copy(acc.at[pl.ds(row0 + off, n)], stage.at[pl.ds(0, n)])
    pltpu.sync_copy(stage.at[pl.ds(0, n)], out_at(off, n))


def _make_segsum(n_edges):
  epw = n_edges // NW
  nch = epw // CH

  def body(table, src, dst, zrows, out, acc, idx_s, idx_d, rows, sem):
    cid = lax.axis_index("c")
    sid = lax.axis_index("s")
    wid = sid * NC + cid
    r0 = sid * ZRM
    pltpu.sync_copy(zrows, rows)
    _zero_spmem(rows, acc, r0, ZRM)
    plsc.subcore_barrier()

    base = wid * epw

    def step(j, carry):
      o = base + j * CH
      pltpu.sync_copy(src.at[pl.ds(o, CH)], idx_s)
      pltpu.async_copy(table.at[idx_s], rows, sem).wait()
      pltpu.sync_copy(dst.at[pl.ds(o, CH)], idx_d)
      pltpu.sync_copy(rows, acc.at[idx_d], add=True)
      return carry

    lax.fori_loop(0, nch, step, 0, unroll=False)
    plsc.subcore_barrier()
    _spmem_to_hbm(acc, rows,
                  lambda off, n: out.at[cid, pl.ds(r0 + off, n)], r0, ZRM)

  return pl.kernel(
      body,
      out_type=jax.ShapeDtypeStruct((NC, NMP, HD), jnp.float32),
      mesh=_MESH,
      scratch_types=[
          pltpu.VMEM_SHARED((NMP, HD), jnp.float32),
          pltpu.VMEM((CH,), jnp.int32),
          pltpu.VMEM((CH,), jnp.int32),
          pltpu.VMEM((CH, HD), jnp.float32),
          pltpu.SemaphoreType.DMA,
      ])


def _make_deg_mesh(n_edges):
  epw = n_edges // NW
  nch = epw // CH

  def body(dst, zrows, ones, out, acc, idx_d, buf):
    cid = lax.axis_index("c")
    sid = lax.axis_index("s")
    wid = sid * NC + cid
    r0 = sid * ZRM
    pltpu.sync_copy(zrows, buf)
    _zero_spmem(buf, acc, r0, ZRM)
    pltpu.sync_copy(ones, buf)
    plsc.subcore_barrier()

    base = wid * epw

    def step(j, carry):
      o = base + j * CH
      pltpu.sync_copy(dst.at[pl.ds(o, CH)], idx_d)
      pltpu.sync_copy(buf, acc.at[idx_d], add=True)
      return carry

    lax.fori_loop(0, nch, step, 0, unroll=False)
    plsc.subcore_barrier()
    _spmem_to_hbm(acc, buf,
                  lambda off, n: out.at[cid, pl.ds(r0 + off, n)], r0, ZRM)

  return pl.kernel(
      body,
      out_type=jax.ShapeDtypeStruct((NC, NMP, HD), jnp.float32),
      mesh=_MESH,
      scratch_types=[
          pltpu.VMEM_SHARED((NMP, HD), jnp.float32),
          pltpu.VMEM((CH,), jnp.int32),
          pltpu.VMEM((CH, HD), jnp.float32),
      ])


def _make_m2g(with_gather):
  ept = EM2G // NS
  nch = ept // CH

  def body_sum(table, src, eidx, zrows, out, acc, idx_s, idx_d, rows,
               sem):
    cid = lax.axis_index("c")
    sid = lax.axis_index("s")
    base = sid * ept

    for cc in range(2):
      chunk = cid * 2 + cc
      pltpu.sync_copy(zrows, rows)
      _zero_spmem(rows, acc, sid * ZRG, ZRG)
      plsc.subcore_barrier()

      def step(j, carry):
        o = base + j * CH
        pltpu.sync_copy(src.at[pl.ds(o, CH)], idx_s)
        pltpu.async_copy(table.at[idx_s], rows, sem).wait()
        pltpu.sync_copy(eidx.at[chunk, pl.ds(o, CH)], idx_d)
        pltpu.sync_copy(rows, acc.at[idx_d], add=True)
        return carry

      lax.fori_loop(0, nch, step, 0, unroll=False)
      plsc.subcore_barrier()
      lo = chunk * MGC + sid * ORG
      _spmem_to_hbm(acc, rows,
                    lambda off, n: out.at[pl.ds(lo + off, n)],
                    sid * ORG, ORG)
      plsc.subcore_barrier()

  def body_deg(eidx, zrows, ones, out, acc, idx_d, buf):
    cid = lax.axis_index("c")
    sid = lax.axis_index("s")
    base = sid * ept

    for cc in range(2):
      chunk = cid * 2 + cc
      pltpu.sync_copy(zrows, buf)
      _zero_spmem(buf, acc, sid * ZRG, ZRG)
      pltpu.sync_copy(ones, buf)
      plsc.subcore_barrier()

      def step(j, carry):
        o = base + j * CH
        pltpu.sync_copy(eidx.at[chunk, pl.ds(o, CH)], idx_d)
        pltpu.sync_copy(buf, acc.at[idx_d], add=True)
        return carry

      lax.fori_loop(0, nch, step, 0, unroll=False)
      plsc.subcore_barrier()
      lo = chunk * MGC + sid * ORG
      _spmem_to_hbm(acc, buf,
                    lambda off, n: out.at[pl.ds(lo + off, n)],
                    sid * ORG, ORG)
      plsc.subcore_barrier()

  if with_gather:
    return pl.kernel(
        body_sum,
        out_type=jax.ShapeDtypeStruct((NG, HD), jnp.float32),
        mesh=_MESH,
        scratch_types=[
            pltpu.VMEM_SHARED((MACC, HD), jnp.float32),
            pltpu.VMEM((CH,), jnp.int32),
            pltpu.VMEM((CH,), jnp.int32),
            pltpu.VMEM((CH, HD), jnp.float32),
            pltpu.SemaphoreType.DMA,
        ])
  return pl.kernel(
      body_deg,
      out_type=jax.ShapeDtypeStruct((NG, HD), jnp.float32),
      mesh=_MESH,
      scratch_types=[
          pltpu.VMEM_SHARED((MACC, HD), jnp.float32),
          pltpu.VMEM((CH,), jnp.int32),
          pltpu.VMEM((CH, HD), jnp.float32),
      ])


def _enc_body(x_ref, w_ref, b_ref, o_ref):
  o_ref[...] = jnp.maximum(
      jnp.dot(x_ref[...], w_ref[...], preferred_element_type=jnp.float32)
      + b_ref[...], 0.0)


def _mesh_enc_body(agg_ref, deg_ref, wg_ref, wm_ref, b_ref, o_ref):
  s = agg_ref[0] + agg_ref[1]
  d = deg_ref[0, :, :1] + deg_ref[1, :, :1]
  h = s / jnp.maximum(d, 1.0)
  w = jnp.dot(wg_ref[...], wm_ref[...], preferred_element_type=jnp.float32)
  o_ref[...] = jnp.maximum(
      jnp.dot(h, w, preferred_element_type=jnp.float32) + b_ref[...], 0.0)


def _proc_body(h_ref, a_ref, w_ref, b_ref, o_ref):
  a = a_ref[0] + a_ref[1]
  o_ref[...] = h_ref[...] + jnp.maximum(
      jnp.dot(a, w_ref[...], preferred_element_type=jnp.float32)
      + b_ref[...], 0.0)


def _dec_body(hg_ref, a_ref, deg_ref, w1_ref, b1_ref, w2_ref, b2_ref,
              o_ref):
  d = jnp.maximum(deg_ref[:, :1], 1.0)
  t = jnp.maximum(
      jnp.dot(hg_ref[...], w1_ref[0], preferred_element_type=jnp.float32)
      + jnp.dot(a_ref[...] / d, w1_ref[1],
                preferred_element_type=jnp.float32)
      + b1_ref[...], 0.0)
  o_ref[...] = jnp.dot(t, w2_ref[...],
                       preferred_element_type=jnp.float32) + b2_ref[...]


_segsum_g2m = _make_segsum(EG2M)
_segsum_mesh = _make_segsum(EMESH)
_deg_mesh = _make_deg_mesh(EG2M)
_m2g_sum = _make_m2g(True)
_m2g_deg = _make_m2g(False)


@jax.jit
def kernel(X, g2m_src, g2m_dst, mesh_edge_index, m2g_src, m2g_dst, W_ge,
           b_ge, W_g2m, W_me, b_me, W_p1, b_p1, W_p2, b_p2, W_d1, b_d1,
           W_d2, b_d2):
  x = X[0]
  b_ge2 = b_ge[None]
  b_me2 = b_me[None]
  b_d12 = b_d1[None]
  b_d22 = b_d2[None]
  w_d1s = W_d1.reshape(2, HD, HD)

  zrows = jnp.zeros((CH, HD), jnp.float32)
  ones_r = jnp.ones((CH, HD), jnp.float32)

  spread = jnp.arange(EM2G, dtype=jnp.int32) & 127
  chunk_of = lax.shift_right_logical(m2g_dst, 13)
  local = jnp.bitwise_and(m2g_dst, MGC - 1)
  eidx = jnp.where(
      chunk_of[None, :] == jnp.arange(4, dtype=jnp.int32)[:, None],
      local[None, :], MGC + spread[None, :])

  nb = 8
  h_grid = pl.pallas_call(
      _enc_body,
      grid=(nb,),
      in_specs=[
          pl.BlockSpec((NG // nb, DF), lambda i: (i, 0)),
          pl.BlockSpec((DF, HD), lambda i: (0, 0)),
          pl.BlockSpec((1, HD), lambda i: (0, 0)),
      ],
      out_specs=pl.BlockSpec((NG // nb, HD), lambda i: (i, 0)),
      out_shape=jax.ShapeDtypeStruct((NG, HD), jnp.float32),
  )(x, W_ge, b_ge2)

  agg = _segsum_g2m(h_grid, g2m_src, g2m_dst, zrows)
  deg = _deg_mesh(g2m_dst, zrows, ones_r)

  h_mesh = pl.pallas_call(
      _mesh_enc_body,
      out_shape=jax.ShapeDtypeStruct((NMP, HD), jnp.float32),
  )(agg, deg, W_g2m, W_me, b_me2)

  msrc = mesh_edge_index[0]
  mdst = mesh_edge_index[1]
  for w, b in ((W_p1, b_p1[None]), (W_p2, b_p2[None])):
    a = _segsum_mesh(h_mesh, msrc, mdst, zrows)
    h_mesh = pl.pallas_call(
        _proc_body,
        out_shape=jax.ShapeDtypeStruct((NMP, HD), jnp.float32),
    )(h_mesh, a, w, b)

  a3 = _m2g_sum(h_mesh, m2g_src, eidx, zrows)
  deg_g = _m2g_deg(eidx, zrows, ones_r)

  out = pl.pallas_call(
      _dec_body,
      grid=(nb,),
      in_specs=[
          pl.BlockSpec((NG // nb, HD), lambda i: (i, 0)),
          pl.BlockSpec((NG // nb, HD), lambda i: (i, 0)),
          pl.BlockSpec((NG // nb, HD), lambda i: (i, 0)),
          pl.BlockSpec((2, HD, HD), lambda i: (0, 0, 0)),
          pl.BlockSpec((1, HD), lambda i: (0, 0)),
          pl.BlockSpec((HD, DF), lambda i: (0, 0)),
          pl.BlockSpec((1, DF), lambda i: (0, 0)),
      ],
      out_specs=pl.BlockSpec((NG // nb, DF), lambda i: (i, 0)),
      out_shape=jax.ShapeDtypeStruct((NG, DF), jnp.float32),
  )(h_grid, a3, deg_g, w_d1s, b_d12, W_d2, b_d22)

  return out[None]
